# baseline (device time: 19540 ns/iter reference)
import jax
import jax.numpy as jnp
from jax import lax
from jax.experimental import pallas as pl
from jax.experimental.pallas import tpu as pltpu

QR = 256
CK = 32
NC = QR // CK
GX, GY, GZ = 104, 72, 80
OY = GX
OZ = GX + GY
SRC_Y = (3, 4, 5)
SRC_Z = (5, 6, 7)
ORD = (5, 3, 4, 6, 7, 2, 1, 0)


def kernel(x):
    _, m, n = x.shape
    cols = n // 2
    x = x.reshape(m, n)

    def body(x_ref, out_ref, xl, xo, xg, yo, yg, zo, zg,
             lsem, xs, xr, ys, yr, zs, zr):
        mx = lax.axis_index("x")
        myy = lax.axis_index("y")
        mz = lax.axis_index("z")
        py = lax.rem(myy, 2)
        pz = lax.rem(mz, 2)
        partner = (1 - mx, myy, mz)
        ymate = (mx, myy + 1 - 2 * py, mz)
        zmate = (mx, myy, mz + 1 - 2 * pz)

        barrier_sem = pltpu.get_barrier_semaphore()
        for nbr in (partner, ymate, zmate):
            pl.semaphore_signal(
                barrier_sem, inc=1,
                device_id=nbr, device_id_type=pl.DeviceIdType.MESH,
            )
        pl.semaphore_wait(barrier_sem, 3)

        pcol0 = (1 - mx) * cols
        mcol0 = mx * cols

        lcopy = pltpu.make_async_copy(
            x_ref.at[pl.ds(0, m), pl.ds(mcol0, cols)], xl, lsem,
        )
        lcopy.start()
        q_me = (2 * py + pz) * QR
        q_y = (2 * (1 - py) + pz) * QR
        q_z = (2 * py + (1 - pz)) * QR
        q_g = (2 * (1 - py) + (1 - pz)) * QR

        def send(src, dst, ssem, rsem, dev):
            r = pltpu.make_async_remote_copy(
                src_ref=src, dst_ref=dst, send_sem=ssem, recv_sem=rsem,
                device_id=dev, device_id_type=pl.DeviceIdType.MESH,
            )
            r.start()
            return r

        def add(row0, rows, buf):
            out_ref[pl.ds(row0, rows), :] = xl[pl.ds(row0, rows), :] + buf

        A = {}
        for k in ORD:
            A[k] = send(
                x_ref.at[pl.ds(q_me + k * CK, CK), pl.ds(pcol0, cols)],
                xo.at[pl.ds(k * CK, CK)],
                xs.at[k], xr.at[k], partner,
            )
        Ag = send(
            x_ref.at[pl.ds(q_g, GX), pl.ds(pcol0, cols)],
            xg, xs.at[NC], xr.at[NC], partner,
        )

        B, Cc = {}, {}
        for k in ORD:
            A[k].wait_recv()
            B[k] = send(
                xo.at[pl.ds(k * CK, CK)], yo.at[pl.ds(k * CK, CK)],
                ys.at[k], yr.at[k], ymate,
            )
            Cc[k] = send(
                xo.at[pl.ds(k * CK, CK)], zo.at[pl.ds(k * CK, CK)],
                zs.at[k], zr.at[k], zmate,
            )

        for k in SRC_Y:
            Cc[k].wait_recv()
        Bg = send(zo.at[pl.ds(OY, GY)], yg, ys.at[NC], yr.at[NC], ymate)

        for k in SRC_Z:
            B[k].wait_recv()
        Cg = send(yo.at[pl.ds(OZ, GZ)], zg, zs.at[NC], zr.at[NC], zmate)

        lcopy.wait()
        for k in range(NC):
            add(q_me + k * CK, CK, xo[pl.ds(k * CK, CK)])
        Ag.wait_recv()
        add(q_g, GX, xg[...])
        for k in SRC_Y:
            add(q_z + k * CK, CK, zo[pl.ds(k * CK, CK)])
        for k in SRC_Z:
            add(q_y + k * CK, CK, yo[pl.ds(k * CK, CK)])
        for k in ORD:
            if k not in SRC_Z:
                B[k].wait_recv()
                add(q_y + k * CK, CK, yo[pl.ds(k * CK, CK)])
            if k not in SRC_Y:
                Cc[k].wait_recv()
                add(q_z + k * CK, CK, zo[pl.ds(k * CK, CK)])

        Bg.wait_recv()
        add(q_g + OY, GY, yg[...])
        Cg.wait_recv()
        add(q_g + OZ, GZ, zg[...])

        for r in list(A.values()) + list(B.values()) + list(Cc.values()) + [Ag, Bg, Cg]:
            r.wait_send()

    return pl.pallas_call(
        body,
        out_shape=jax.ShapeDtypeStruct((m, cols), jnp.float32),
        in_specs=[pl.BlockSpec(memory_space=pltpu.MemorySpace.HBM)],
        out_specs=pl.BlockSpec(memory_space=pltpu.VMEM),
        scratch_shapes=[
            pltpu.VMEM((m, cols), jnp.float32),
            pltpu.VMEM((QR, cols), jnp.float32),
            pltpu.VMEM((GX, cols), jnp.float32),
            pltpu.VMEM((QR, cols), jnp.float32),
            pltpu.VMEM((GY, cols), jnp.float32),
            pltpu.VMEM((QR, cols), jnp.float32),
            pltpu.VMEM((GZ, cols), jnp.float32),
            pltpu.SemaphoreType.DMA,
            pltpu.SemaphoreType.DMA((NC + 1,)),
            pltpu.SemaphoreType.DMA((NC + 1,)),
            pltpu.SemaphoreType.DMA((NC + 1,)),
            pltpu.SemaphoreType.DMA((NC + 1,)),
            pltpu.SemaphoreType.DMA((NC + 1,)),
            pltpu.SemaphoreType.DMA((NC + 1,)),
        ],
        compiler_params=pltpu.CompilerParams(collective_id=0),
    )(x)


# device time: 6377 ns/iter; 3.0641x vs baseline; 3.0641x over previous
import jax
import jax.numpy as jnp
from jax import lax
from jax.experimental import pallas as pl
from jax.experimental.pallas import tpu as pltpu

QR = 256
CK = 64
NC = QR // CK
GX, GY, GZ = 104, 72, 80
OY = GX
OZ = GX + GY
SRC_Y = (1, 2)
SRC_Z = (2, 3)
ORD = (2, 1, 3, 0)


def kernel(x):
    _, m, n = x.shape
    cols = n // 2
    x = x.reshape(m, n)

    def body(x_ref, out_ref, xo, xg, yo, yg, zo, zg, xs, xr, ys, yr, zs, zr):
        mx = lax.axis_index("x")
        myy = lax.axis_index("y")
        mz = lax.axis_index("z")
        py = lax.rem(myy, 2)
        pz = lax.rem(mz, 2)
        partner = (1 - mx, myy, mz)
        ymate = (mx, myy + 1 - 2 * py, mz)
        zmate = (mx, myy, mz + 1 - 2 * pz)

        barrier_sem = pltpu.get_barrier_semaphore()
        for nbr in (partner, ymate, zmate):
            pl.semaphore_signal(
                barrier_sem, inc=1,
                device_id=nbr, device_id_type=pl.DeviceIdType.MESH,
            )
        pl.semaphore_wait(barrier_sem, 3)

        pcol0 = (1 - mx) * cols
        mcol0 = mx * cols
        q_me = (2 * py + pz) * QR
        q_y = (2 * (1 - py) + pz) * QR
        q_z = (2 * py + (1 - pz)) * QR
        q_g = (2 * (1 - py) + (1 - pz)) * QR

        def send(src, dst, ssem, rsem, dev):
            r = pltpu.make_async_remote_copy(
                src_ref=src, dst_ref=dst, send_sem=ssem, recv_sem=rsem,
                device_id=dev, device_id_type=pl.DeviceIdType.MESH,
            )
            r.start()
            return r

        def add(row0, rows, buf):
            out_ref[pl.ds(row0, rows), :] = (
                x_ref[pl.ds(row0, rows), pl.ds(mcol0, cols)] + buf
            )

        A = {}
        for k in ORD:
            A[k] = send(
                x_ref.at[pl.ds(q_me + k * CK, CK), pl.ds(pcol0, cols)],
                xo.at[pl.ds(k * CK, CK)],
                xs.at[k], xr.at[k], partner,
            )
        Ag = send(
            x_ref.at[pl.ds(q_g, GX), pl.ds(pcol0, cols)],
            xg, xs.at[NC], xr.at[NC], partner,
        )

        B, Cc = {}, {}
        for k in ORD:
            A[k].wait_recv()
            B[k] = send(
                xo.at[pl.ds(k * CK, CK)], yo.at[pl.ds(k * CK, CK)],
                ys.at[k], yr.at[k], ymate,
            )
            Cc[k] = send(
                xo.at[pl.ds(k * CK, CK)], zo.at[pl.ds(k * CK, CK)],
                zs.at[k], zr.at[k], zmate,
            )

        for k in SRC_Y:
            Cc[k].wait_recv()
        Bg = send(zo.at[pl.ds(OY, GY)], yg, ys.at[NC], yr.at[NC], ymate)

        for k in SRC_Z:
            B[k].wait_recv()
        Cg = send(yo.at[pl.ds(OZ, GZ)], zg, zs.at[NC], zr.at[NC], zmate)

        for k in range(NC):
            add(q_me + k * CK, CK, xo[pl.ds(k * CK, CK)])
        Ag.wait_recv()
        add(q_g, GX, xg[...])
        for k in SRC_Y:
            add(q_z + k * CK, CK, zo[pl.ds(k * CK, CK)])
        for k in SRC_Z:
            add(q_y + k * CK, CK, yo[pl.ds(k * CK, CK)])
        for k in ORD:
            if k not in SRC_Z:
                B[k].wait_recv()
                add(q_y + k * CK, CK, yo[pl.ds(k * CK, CK)])
            if k not in SRC_Y:
                Cc[k].wait_recv()
                add(q_z + k * CK, CK, zo[pl.ds(k * CK, CK)])

        Bg.wait_recv()
        add(q_g + OY, GY, yg[...])
        Cg.wait_recv()
        add(q_g + OZ, GZ, zg[...])

        for r in list(A.values()) + list(B.values()) + list(Cc.values()) + [Ag, Bg, Cg]:
            r.wait_send()

    return pl.pallas_call(
        body,
        out_shape=jax.ShapeDtypeStruct((m, cols), jnp.float32),
        in_specs=[pl.BlockSpec(memory_space=pltpu.VMEM)],
        out_specs=pl.BlockSpec(memory_space=pltpu.VMEM),
        scratch_shapes=[
            pltpu.VMEM((QR, cols), jnp.float32),
            pltpu.VMEM((GX, cols), jnp.float32),
            pltpu.VMEM((QR, cols), jnp.float32),
            pltpu.VMEM((GY, cols), jnp.float32),
            pltpu.VMEM((QR, cols), jnp.float32),
            pltpu.VMEM((GZ, cols), jnp.float32),
            pltpu.SemaphoreType.DMA((NC + 1,)),
            pltpu.SemaphoreType.DMA((NC + 1,)),
            pltpu.SemaphoreType.DMA((NC + 1,)),
            pltpu.SemaphoreType.DMA((NC + 1,)),
            pltpu.SemaphoreType.DMA((NC + 1,)),
            pltpu.SemaphoreType.DMA((NC + 1,)),
        ],
        compiler_params=pltpu.CompilerParams(collective_id=0),
    )(x)
